# baseline (speedup 1.0000x reference)
import jax
import jax.numpy as jnp
from jax.experimental import pallas as pl
from jax.experimental.pallas import tpu as pltpu

B, T, D = 2, 2048, 2048
E = 8
O = 2048
I = D // E
C = 256
TB = 256
NT = T // TB
NS = B * NT
P0 = E


def _cb_copy(cb_hbm, cb_vmem, sems, m):
    bm = m // NT
    tm = m % NT
    return pltpu.make_async_copy(
        cb_hbm.at[bm, pl.ds(tm * TB, TB)],
        cb_vmem.at[m],
        sems.at[m],
    )


def _fused_kernel(w_ref, b_ref, x_ref, dm_ref, cb_hbm, o_ref,
                  ws_scr, bs_scr, s_scr, cb_vmem, sems):
    s = pl.program_id(0)

    @pl.when(s < NS)
    def _cb_prefetch():
        _cb_copy(cb_hbm, cb_vmem, sems, s).start()

    @pl.when(s < P0)
    def _w_phase():
        wblk = w_ref[...]
        acc = wblk[:, 0:I]
        for j in range(1, E):
            acc = acc + wblk[:, j * I:(j + 1) * I]
        ws_scr[pl.ds(s, 1), :] = jnp.sum(acc, axis=0, keepdims=True)

        @pl.when(s == 0)
        def _():
            bs_scr[0, 0] = jnp.sum(b_ref[...])

    @pl.when((s >= P0) & (s < P0 + NS))
    def _xdm_phase():
        j = s - P0
        bb = j // NT
        init = (j % NT) == 0
        xb = x_ref[0]
        dmb = dm_ref[0]
        for e in range(E):
            we = ws_scr[e:e + 1, :]
            p_e = jnp.sum(xb[:, e * I:(e + 1) * I] * we, axis=1,
                          keepdims=True)
            contrib = jnp.sum(dmb[:, e, :] * p_e, axis=0,
                              keepdims=True)
            row = bb * E + e
            prev = jnp.where(init, bs_scr[0, 0], s_scr[pl.ds(row, 1), :])
            s_scr[pl.ds(row, 1), :] = prev + contrib

    @pl.when(s >= P0 + NS)
    def _cb_phase():
        j = s - P0 - NS
        bb = j // NT
        _cb_copy(cb_hbm, cb_vmem, sems, j).wait()
        cbb = cb_vmem[j]
        acc = cbb[:, 0, :] * s_scr[pl.ds(bb * E, 1), :]
        for e in range(1, E):
            acc = acc + cbb[:, e, :] * s_scr[pl.ds(bb * E + e, 1), :]
        o_ref[...] = jnp.sum(acc, axis=1).reshape(1, 1, TB)


def kernel(x, combine_array, dispatch_mask, W, b):
    b2 = b.reshape(E, O // E)

    def w_idx(s):
        return (jnp.minimum(s, P0 - 1), 0)

    def dm_idx(s):
        j = jnp.clip(s - P0, 0, NS - 1)
        return (j // NT, j % NT, 0, 0)

    def x_idx(s):
        j = jnp.clip(s - P0, 0, NS - 1)
        return (j // NT, j % NT, 0)

    def out_idx(s):
        j = jnp.clip(s - P0 - NS, 0, NS - 1)
        return (j // NT, 0, j % NT)

    out = pl.pallas_call(
        _fused_kernel,
        grid=(P0 + 2 * NS,),
        in_specs=[
            pl.BlockSpec((O // E, D), w_idx),
            pl.BlockSpec((E, O // E), lambda s: (0, 0)),
            pl.BlockSpec((1, TB, D), x_idx),
            pl.BlockSpec((1, TB, E, C), dm_idx),
            pl.BlockSpec(memory_space=pl.ANY),
        ],
        out_specs=pl.BlockSpec((1, 1, TB), out_idx),
        out_shape=jax.ShapeDtypeStruct((B, 1, T), jnp.float32),
        scratch_shapes=[
            pltpu.VMEM((E, I), jnp.float32),
            pltpu.SMEM((1, 1), jnp.float32),
            pltpu.VMEM((B * E, C), jnp.float32),
            pltpu.VMEM((NS, TB, E, C), jnp.float32),
            pltpu.SemaphoreType.DMA((NS,)),
        ],
    )(W, b2, x, dispatch_mask, combine_array)

    return out.reshape(B, T)


# speedup vs baseline: 1.0712x; 1.0712x over previous
import jax
import jax.numpy as jnp
from jax.experimental import pallas as pl
from jax.experimental.pallas import tpu as pltpu

B, T, D = 2, 2048, 2048
E = 8
O = 2048
I = D // E
C = 256
TB = 512
NT = T // TB
NS = B * NT
TB2 = 1024
NT2 = T // TB2
NS2 = B * NT2
P0 = 4


def _fused_kernel(w_ref, b_ref, x_ref, dm_ref, cb_ref, o_ref,
                  ws_scr, bs_scr, s_scr):
    s = pl.program_id(0)

    @pl.when(s < P0)
    def _w_phase():
        wblk = w_ref[...]
        for h in range(2):
            sub = wblk[h * 256:(h + 1) * 256]
            acc = sub[:, 0:I]
            for j in range(1, E):
                acc = acc + sub[:, j * I:(j + 1) * I]
            ws_scr[pl.ds(2 * s + h, 1), :] = jnp.sum(acc, axis=0,
                                                     keepdims=True)

        @pl.when(s == 0)
        def _():
            bs_scr[0, 0] = jnp.sum(b_ref[...])

    @pl.when((s >= P0) & (s < P0 + NS))
    def _xdm_phase():
        j = s - P0
        bb = j // NT
        init = (j % NT) == 0
        xb = x_ref[0]
        dmb = dm_ref[0]
        for e in range(E):
            we = ws_scr[e:e + 1, :]
            p_e = jnp.sum(xb[:, e * I:(e + 1) * I] * we, axis=1,
                          keepdims=True)
            contrib = jnp.sum(dmb[:, e, :] * p_e, axis=0,
                              keepdims=True)
            row = bb * E + e
            prev = jnp.where(init, bs_scr[0, 0], s_scr[pl.ds(row, 1), :])
            s_scr[pl.ds(row, 1), :] = prev + contrib

    @pl.when(s >= P0 + NS)
    def _cb_phase():
        j = s - P0 - NS
        bb = j // NT2
        cbb = cb_ref[0]
        acc = cbb[:, 0, :] * s_scr[pl.ds(bb * E, 1), :]
        for e in range(1, E):
            acc = acc + cbb[:, e, :] * s_scr[pl.ds(bb * E + e, 1), :]
        o_ref[...] = jnp.sum(acc, axis=1).reshape(1, 1, TB2)


def kernel(x, combine_array, dispatch_mask, W, b):
    b2 = b.reshape(E, O // E)

    def w_idx(s):
        return (jnp.minimum(s, P0 - 1), 0)

    def dm_idx(s):
        j = jnp.clip(s - P0, 0, NS - 1)
        return (j // NT, j % NT, 0, 0)

    def cb_idx(s):
        j = jnp.clip(s - P0 - NS, 0, NS2 - 1)
        return (j // NT2, j % NT2, 0, 0)

    def x_idx(s):
        j = jnp.clip(s - P0, 0, NS - 1)
        return (j // NT, j % NT, 0)

    def out_idx(s):
        j = jnp.clip(s - P0 - NS, 0, NS2 - 1)
        return (j // NT2, 0, j % NT2)

    out = pl.pallas_call(
        _fused_kernel,
        grid=(P0 + NS + NS2,),
        in_specs=[
            pl.BlockSpec((O // P0, D), w_idx),
            pl.BlockSpec((E, O // E), lambda s: (0, 0)),
            pl.BlockSpec((1, TB, D), x_idx),
            pl.BlockSpec((1, TB, E, C), dm_idx),
            pl.BlockSpec((1, TB2, E, C), cb_idx),
        ],
        out_specs=pl.BlockSpec((1, 1, TB2), out_idx),
        out_shape=jax.ShapeDtypeStruct((B, 1, T), jnp.float32),
        scratch_shapes=[
            pltpu.VMEM((E, I), jnp.float32),
            pltpu.SMEM((1, 1), jnp.float32),
            pltpu.VMEM((B * E, C), jnp.float32),
        ],
    )(W, b2, x, dispatch_mask, combine_array)

    return out.reshape(B, T)
